# Initial kernel scaffold; baseline (speedup 1.0000x reference)
import jax
import jax.numpy as jnp
from jax.experimental import pallas as pl


def kernel(x, edge_index, W1, b1, W2, b2):
    raise NotImplementedError("write your pallas kernel here")


# speedup vs baseline: 25.0014x; 25.0014x over previous
import functools

import jax
import jax.numpy as jnp
from jax import lax
from jax.experimental import pallas as pl
from jax.experimental.pallas import tpu as pltpu
from jax.experimental.pallas import tpu_sc as plsc

NC = 2
NS = 16
NW = NC * NS


def _make_deg_kernel(NP, CHUNKS, K):
    rpt = NP // NS

    def body(dst_hbm, zrow_hbm, out_hbm, dstv, onesv, accsh):
        c = lax.axis_index("c")
        s = lax.axis_index("s")
        wid = s * NC + c
        pltpu.sync_copy(dst_hbm.at[wid], dstv)
        for j in range(K // 16):
            onesv[pl.ds(j * 16, 16)] = jnp.full((16,), 1.0, jnp.float32)
        pltpu.sync_copy(zrow_hbm, accsh.at[pl.ds(s * rpt, rpt)])
        plsc.subcore_barrier()

        def step(i, carry):
            pltpu.sync_copy(onesv, accsh.at[dstv.at[i]], add=True)
            return carry

        lax.fori_loop(0, CHUNKS, step, 0)
        plsc.subcore_barrier()
        pltpu.sync_copy(accsh.at[pl.ds(s * rpt, rpt)],
                        out_hbm.at[pl.ds(c * NP + s * rpt, rpt)])

    return pl.kernel(
        body,
        out_type=jax.ShapeDtypeStruct((NC * NP,), jnp.float32),
        mesh=plsc.VectorSubcoreMesh(core_axis_name="c", subcore_axis_name="s"),
        compiler_params=pltpu.CompilerParams(use_tc_tiling_on_sc=False),
        scratch_types=[
            pltpu.VMEM((CHUNKS, K), jnp.int32),
            pltpu.VMEM((K,), jnp.float32),
            pltpu.VMEM_SHARED((NP,), jnp.float32),
        ],
    )


def _make_agg_kernel(NP, F, CHUNKS, K):
    rpt = NP // NS

    def body(y_hbm, src_hbm, dst_hbm, zrows_hbm, out_hbm,
             srcv, dstv, rows, accsh, gsem):
        c = lax.axis_index("c")
        s = lax.axis_index("s")
        wid = s * NC + c
        pltpu.sync_copy(src_hbm.at[wid], srcv)
        pltpu.sync_copy(dst_hbm.at[wid], dstv)
        pltpu.sync_copy(zrows_hbm, accsh.at[pl.ds(s * rpt, rpt)])
        plsc.subcore_barrier()

        def step(i, carry):
            pltpu.async_copy(y_hbm.at[srcv.at[i]], rows, gsem).wait()
            pltpu.sync_copy(rows, accsh.at[dstv.at[i]], add=True)
            return carry

        lax.fori_loop(0, CHUNKS, step, 0)
        plsc.subcore_barrier()
        pltpu.sync_copy(accsh.at[pl.ds(s * rpt, rpt)],
                        out_hbm.at[c, pl.ds(s * rpt, rpt)])

    return pl.kernel(
        body,
        out_type=jax.ShapeDtypeStruct((NC, NP, F), jnp.float32),
        mesh=plsc.VectorSubcoreMesh(core_axis_name="c", subcore_axis_name="s"),
        compiler_params=pltpu.CompilerParams(use_tc_tiling_on_sc=False),
        scratch_types=[
            pltpu.VMEM((CHUNKS, K), jnp.int32),
            pltpu.VMEM((CHUNKS, K), jnp.int32),
            pltpu.VMEM((K, F), jnp.float32),
            pltpu.VMEM_SHARED((NP, F), jnp.float32),
            pltpu.SemaphoreType.DMA,
        ],
    )


def _scale_body(d0, d1, x, y, dinv):
    deg = d0[...] + d1[...] + 1.0
    r = lax.rsqrt(deg)
    y[...] = x[...] * r
    dinv[...] = r


def _mm_body(p0, p1, y1, dinv, w1, b1, w2, z):
    agg = (p0[...] + p1[...] + y1[...]) * dinv[...]
    h = jnp.dot(agg, w1[...], preferred_element_type=jnp.float32) + b1[...]
    h = jnp.maximum(h, 0.0)
    z[...] = jnp.dot(h, w2[...], preferred_element_type=jnp.float32) * dinv[...]


def _out_body(q0, q1, z, dinv, b2, o):
    o[...] = (q0[...] + q1[...] + z[...]) * dinv[...] + b2[...]


def kernel(x, edge_index, W1, b1, W2, b2):
    N, Fin = x.shape
    H = W1.shape[1]
    C = W2.shape[1]
    E = edge_index.shape[1]

    K = 80
    CHUNKS = E // (NW * K)
    NP = ((N + NS * 128 - 1) // (NS * 128)) * (NS * 128)
    BN = 1000

    src3 = edge_index[0].reshape(NW, CHUNKS, K)
    dst3 = edge_index[1].reshape(NW, CHUNKS, K)
    rpt = NP // NS
    zrow = jnp.zeros((rpt,), jnp.float32)
    zrows1 = jnp.zeros((rpt, Fin), jnp.float32)
    zrows2 = jnp.zeros((rpt, C), jnp.float32)

    dp = _make_deg_kernel(NP, CHUNKS, K)(dst3, zrow).reshape(NC, NP)
    grid = (N // BN,)
    row = lambda i: (i, 0)
    y1, dinv = pl.pallas_call(
        _scale_body,
        grid=grid,
        in_specs=[pl.BlockSpec((BN, 1), row), pl.BlockSpec((BN, 1), row),
                  pl.BlockSpec((BN, Fin), row)],
        out_specs=[pl.BlockSpec((BN, Fin), row), pl.BlockSpec((BN, 1), row)],
        out_shape=[jax.ShapeDtypeStruct((N, Fin), jnp.float32),
                   jax.ShapeDtypeStruct((N, 1), jnp.float32)],
    )(dp[0, :N, None], dp[1, :N, None], x)

    p = _make_agg_kernel(NP, Fin, CHUNKS, K)(y1, src3, dst3, zrows1)

    full = lambda i: (0, 0)
    z = pl.pallas_call(
        _mm_body,
        grid=grid,
        in_specs=[pl.BlockSpec((BN, Fin), row), pl.BlockSpec((BN, Fin), row),
                  pl.BlockSpec((BN, Fin), row), pl.BlockSpec((BN, 1), row),
                  pl.BlockSpec((Fin, H), full), pl.BlockSpec((1, H), full),
                  pl.BlockSpec((H, C), full)],
        out_specs=pl.BlockSpec((BN, C), row),
        out_shape=jax.ShapeDtypeStruct((N, C), jnp.float32),
    )(p[0, :N], p[1, :N], y1, dinv, W1, b1.reshape(1, H), W2)

    q = _make_agg_kernel(NP, C, CHUNKS, K)(z, src3, dst3, zrows2)

    out = pl.pallas_call(
        _out_body,
        grid=grid,
        in_specs=[pl.BlockSpec((BN, C), row), pl.BlockSpec((BN, C), row),
                  pl.BlockSpec((BN, C), row), pl.BlockSpec((BN, 1), row),
                  pl.BlockSpec((1, C), full)],
        out_specs=pl.BlockSpec((BN, C), row),
        out_shape=jax.ShapeDtypeStruct((N, C), jnp.float32),
    )(q[0, :N], q[1, :N], z, dinv, b2.reshape(1, C))
    return out


# speedup vs baseline: 34.2205x; 1.3687x over previous
import jax
import jax.numpy as jnp
from jax import lax
from jax.experimental import pallas as pl
from jax.experimental.pallas import tpu as pltpu
from jax.experimental.pallas import tpu_sc as plsc

NC = 2
NS = 16
NW = NC * NS
K = 128


def _make_deg_kernel(NP, CHUNKS):
    rpt = NP // NS

    def body(dst_hbm, zrow_hbm, out_hbm, dstv, onesv, accsh, ssem):
        c = lax.axis_index("c")
        s = lax.axis_index("s")
        wid = s * NC + c
        pltpu.sync_copy(dst_hbm.at[wid], dstv)
        for j in range(K // 16):
            onesv[pl.ds(j * 16, 16)] = jnp.full((16,), 1.0, jnp.float32)
        pltpu.sync_copy(zrow_hbm, accsh.at[pl.ds(s * rpt, rpt)])
        plsc.subcore_barrier()

        def fire(i, carry):
            pltpu.async_copy(onesv, accsh.at[dstv.at[i]], ssem, add=True)
            return carry

        lax.fori_loop(0, CHUNKS, fire, 0)

        def drain(i, carry):
            pltpu.make_async_copy(onesv, accsh.at[dstv.at[i]], ssem).wait()
            return carry

        lax.fori_loop(0, CHUNKS, drain, 0)
        plsc.subcore_barrier()
        pltpu.sync_copy(accsh.at[pl.ds(s * rpt, rpt)],
                        out_hbm.at[pl.ds(c * NP + s * rpt, rpt)])

    return pl.kernel(
        body,
        out_type=jax.ShapeDtypeStruct((NC * NP,), jnp.float32),
        mesh=plsc.VectorSubcoreMesh(core_axis_name="c", subcore_axis_name="s"),
        compiler_params=pltpu.CompilerParams(use_tc_tiling_on_sc=False),
        scratch_types=[
            pltpu.VMEM((CHUNKS, K), jnp.int32),
            pltpu.VMEM((K,), jnp.float32),
            pltpu.VMEM_SHARED((NP,), jnp.float32),
            pltpu.SemaphoreType.DMA,
        ],
    )


def _make_agg_kernel(NP, F, CHUNKS):
    rpt = NP // NS
    HC = CHUNKS // 4

    def body(y_hbm, src_hbm, dst_hbm, zrows_hbm, out_hbm,
             srcv, dstv, rows0, rows1, accsh,
             gsem0, gsem1, ssem0, ssem1, isem0, isem1, isem2, isem3):
        c = lax.axis_index("c")
        s = lax.axis_index("s")
        wid = s * NC + c
        rows = (rows0, rows1)
        gsem = (gsem0, gsem1)
        ssem = (ssem0, ssem1)
        isem = (isem0, isem1, isem2, isem3)
        pltpu.sync_copy(dst_hbm.at[wid], dstv)
        pltpu.sync_copy(zrows_hbm, accsh.at[pl.ds(s * rpt, rpt)])
        plsc.subcore_barrier()

        for k in range(4):
            pltpu.async_copy(src_hbm.at[wid, k], srcv.at[k], isem[k])
        pltpu.make_async_copy(src_hbm.at[wid, 0], srcv.at[0], isem[0]).wait()
        pltpu.async_copy(y_hbm.at[srcv.at[0]], rows0, gsem0)

        def step(g, carry):
            for jj in range(4):
                j = 4 * g + jj
                b = jj % 2
                sn = (jj + 1) % 4
                bn = (jj + 1) % 2
                pltpu.make_async_copy(y_hbm.at[srcv.at[jj]], rows[b],
                                      gsem[b]).wait()
                pltpu.async_copy(rows[b], accsh.at[dstv.at[j]], ssem[b],
                                 add=True)

                @pl.when(g < HC - 1)
                def _():
                    pltpu.async_copy(src_hbm.at[wid, j + 4], srcv.at[jj],
                                     isem[jj])

                if jj < 3:
                    def _wait_prev_scatter(bn=bn, j=j):
                        pltpu.make_async_copy(
                            rows[bn], accsh.at[dstv.at[j - 1]],
                            ssem[bn]).wait()
                    if jj == 0:
                        pl.when(g > 0)(_wait_prev_scatter)
                    else:
                        _wait_prev_scatter()
                    pltpu.make_async_copy(src_hbm.at[wid, j + 1],
                                          srcv.at[sn], isem[sn]).wait()
                    pltpu.async_copy(y_hbm.at[srcv.at[sn]], rows[bn],
                                     gsem[bn])
                else:
                    @pl.when(g < HC - 1)
                    def _():
                        pltpu.make_async_copy(
                            rows0, accsh.at[dstv.at[j - 1]], ssem0).wait()
                        pltpu.make_async_copy(src_hbm.at[wid, j + 1],
                                              srcv.at[0], isem0).wait()
                        pltpu.async_copy(y_hbm.at[srcv.at[0]], rows0, gsem0)
            return carry

        lax.fori_loop(0, HC, step, 0)
        pltpu.make_async_copy(rows0, accsh.at[dstv.at[CHUNKS - 2]],
                              ssem0).wait()
        pltpu.make_async_copy(rows1, accsh.at[dstv.at[CHUNKS - 1]],
                              ssem1).wait()
        plsc.subcore_barrier()
        pltpu.sync_copy(accsh.at[pl.ds(s * rpt, rpt)],
                        out_hbm.at[c, pl.ds(s * rpt, rpt)])

    return pl.kernel(
        body,
        out_type=jax.ShapeDtypeStruct((NC, NP, F), jnp.float32),
        mesh=plsc.VectorSubcoreMesh(core_axis_name="c", subcore_axis_name="s"),
        compiler_params=pltpu.CompilerParams(use_tc_tiling_on_sc=False),
        scratch_types=[
            pltpu.VMEM((4, K), jnp.int32),
            pltpu.VMEM((CHUNKS, K), jnp.int32),
            pltpu.VMEM((K, F), jnp.float32),
            pltpu.VMEM((K, F), jnp.float32),
            pltpu.VMEM_SHARED((NP, F), jnp.float32),
        ] + [pltpu.SemaphoreType.DMA] * 8,
    )


def _scale_body(d0, d1, x, y, dinv):
    deg = d0[...] + d1[...] + 1.0
    r = lax.rsqrt(deg)
    y[...] = x[...] * r
    dinv[...] = r


def _mm_body(p0, p1, y1, dinv, w1, b1, w2, z):
    agg = (p0[...] + p1[...] + y1[...]) * dinv[...]
    h = jnp.dot(agg, w1[...], preferred_element_type=jnp.float32) + b1[...]
    h = jnp.maximum(h, 0.0)
    z[...] = jnp.dot(h, w2[...], preferred_element_type=jnp.float32) * dinv[...]


def _out_body(q0, q1, z, dinv, b2, o):
    o[...] = (q0[...] + q1[...] + z[...]) * dinv[...] + b2[...]


def kernel(x, edge_index, W1, b1, W2, b2):
    N, Fin = x.shape
    H = W1.shape[1]
    C = W2.shape[1]
    E = edge_index.shape[1]

    NP = ((N + NS * 128 - 1) // (NS * 128)) * (NS * 128)
    CHUNKS = (-(-E // (NW * K)) + 3) // 4 * 4
    EP = NW * CHUNKS * K
    BN = 1024

    pad = EP - E
    padidx = N + (jnp.arange(pad, dtype=edge_index.dtype) % (NP - N))
    src3 = jnp.concatenate([edge_index[0], padidx]).reshape(NW, CHUNKS, K)
    dst3 = jnp.concatenate([edge_index[1], padidx]).reshape(NW, CHUNKS, K)
    xp = jnp.pad(x, ((0, NP - N), (0, 0)))
    rpt = NP // NS
    zrow = jnp.zeros((rpt,), jnp.float32)
    zrows1 = jnp.zeros((rpt, Fin), jnp.float32)
    zrows2 = jnp.zeros((rpt, C), jnp.float32)

    dp = _make_deg_kernel(NP, CHUNKS)(dst3, zrow).reshape(NC, NP)
    grid = (NP // BN,)
    row = lambda i: (i, 0)
    y1, dinv = pl.pallas_call(
        _scale_body,
        grid=grid,
        in_specs=[pl.BlockSpec((BN, 1), row), pl.BlockSpec((BN, 1), row),
                  pl.BlockSpec((BN, Fin), row)],
        out_specs=[pl.BlockSpec((BN, Fin), row), pl.BlockSpec((BN, 1), row)],
        out_shape=[jax.ShapeDtypeStruct((NP, Fin), jnp.float32),
                   jax.ShapeDtypeStruct((NP, 1), jnp.float32)],
    )(dp[0, :, None], dp[1, :, None], xp)

    p = _make_agg_kernel(NP, Fin, CHUNKS)(y1, src3, dst3, zrows1)

    full = lambda i: (0, 0)
    z = pl.pallas_call(
        _mm_body,
        grid=grid,
        in_specs=[pl.BlockSpec((BN, Fin), row), pl.BlockSpec((BN, Fin), row),
                  pl.BlockSpec((BN, Fin), row), pl.BlockSpec((BN, 1), row),
                  pl.BlockSpec((Fin, H), full), pl.BlockSpec((1, H), full),
                  pl.BlockSpec((H, C), full)],
        out_specs=pl.BlockSpec((BN, C), row),
        out_shape=jax.ShapeDtypeStruct((NP, C), jnp.float32),
    )(p[0], p[1], y1, dinv, W1, b1.reshape(1, H), W2)

    q = _make_agg_kernel(NP, C, CHUNKS)(z, src3, dst3, zrows2)

    BO = 1000
    rowo = lambda i: (i, 0)
    out = pl.pallas_call(
        _out_body,
        grid=(N // BO,),
        in_specs=[pl.BlockSpec((BO, C), rowo), pl.BlockSpec((BO, C), rowo),
                  pl.BlockSpec((BO, C), rowo), pl.BlockSpec((BO, 1), rowo),
                  pl.BlockSpec((1, C), full)],
        out_specs=pl.BlockSpec((BO, C), rowo),
        out_shape=jax.ShapeDtypeStruct((N, C), jnp.float32),
    )(q[0], q[1], z, dinv, b2.reshape(1, C))
    return out


# speedup vs baseline: 42.1060x; 1.2304x over previous
import jax
import jax.numpy as jnp
from jax import lax
from jax.experimental import pallas as pl
from jax.experimental.pallas import tpu as pltpu
from jax.experimental.pallas import tpu_sc as plsc

NC = 2
NS = 16
NW = NC * NS
K = 128


def _make_deg_kernel(NP, CHUNKS):
    rpt = NP // NS

    def body(dst_hbm, zrow_hbm, out_hbm, dstv, onesv, accsh, ssem):
        c = lax.axis_index("c")
        s = lax.axis_index("s")
        wid = s * NC + c
        pltpu.sync_copy(dst_hbm.at[wid], dstv)
        for j in range(K // 16):
            onesv[pl.ds(j * 16, 16)] = jnp.full((16,), 1.0, jnp.float32)
        pltpu.sync_copy(zrow_hbm, accsh.at[pl.ds(s * rpt, rpt)])
        plsc.subcore_barrier()

        def fire(i, carry):
            pltpu.async_copy(onesv, accsh.at[dstv.at[i]], ssem, add=True)
            return carry

        lax.fori_loop(0, CHUNKS, fire, 0)

        def drain(i, carry):
            pltpu.make_async_copy(onesv, accsh.at[dstv.at[i]], ssem).wait()
            return carry

        lax.fori_loop(0, CHUNKS, drain, 0)
        plsc.subcore_barrier()
        pltpu.sync_copy(accsh.at[pl.ds(s * rpt, rpt)],
                        out_hbm.at[pl.ds(c * NP + s * rpt, rpt)])

    return pl.kernel(
        body,
        out_type=jax.ShapeDtypeStruct((NC * NP,), jnp.float32),
        mesh=plsc.VectorSubcoreMesh(core_axis_name="c", subcore_axis_name="s"),
        compiler_params=pltpu.CompilerParams(use_tc_tiling_on_sc=False),
        scratch_types=[
            pltpu.VMEM((CHUNKS, K), jnp.int32),
            pltpu.VMEM((K,), jnp.float32),
            pltpu.VMEM_SHARED((NP,), jnp.float32),
            pltpu.SemaphoreType.DMA,
        ],
    )


def _make_agg_kernel(NP, F, CHUNKS, KC):
    rpt = NP // NS
    HC = CHUNKS // 4

    def body(y_hbm, src_hbm, dst_hbm, zrows_hbm, out_hbm,
             srcv, dstv, rows0, rows1, rows2, rows3, accsh, *sems):
        c = lax.axis_index("c")
        s = lax.axis_index("s")
        wid = s * NC + c
        rows = (rows0, rows1, rows2, rows3)
        gsem = sems[0:4]
        ssem = sems[4:8]
        isem = sems[8:12]
        pltpu.sync_copy(dst_hbm.at[wid], dstv)
        pltpu.sync_copy(zrows_hbm, accsh.at[pl.ds(s * rpt, rpt)])
        plsc.subcore_barrier()

        for k in range(4):
            pltpu.async_copy(src_hbm.at[wid, k], srcv.at[k], isem[k])
        for k in range(3):
            pltpu.make_async_copy(src_hbm.at[wid, k], srcv.at[k],
                                  isem[k]).wait()
            pltpu.async_copy(y_hbm.at[srcv.at[k]], rows[k], gsem[k])

        def step(g, carry):
            for jj in range(4):
                j = 4 * g + jj
                q = jj
                qn = (jj + 3) % 4
                pltpu.make_async_copy(y_hbm.at[srcv.at[q]], rows[q],
                                      gsem[q]).wait()
                pltpu.async_copy(rows[q], accsh.at[dstv.at[j]], ssem[q],
                                 add=True)

                @pl.when(g < HC - 1)
                def _():
                    pltpu.async_copy(src_hbm.at[wid, j + 4], srcv.at[q],
                                     isem[q])

                def _issue_next(j=j, q=q, qn=qn):
                    def _ss():
                        pltpu.make_async_copy(
                            rows[qn], accsh.at[dstv.at[j - 1]],
                            ssem[qn]).wait()
                    if jj == 0:
                        pl.when(g > 0)(_ss)
                    else:
                        _ss()
                    pltpu.make_async_copy(src_hbm.at[wid, j + 3],
                                          srcv.at[qn], isem[qn]).wait()
                    pltpu.async_copy(y_hbm.at[srcv.at[qn]], rows[qn],
                                     gsem[qn])

                if jj == 0:
                    _issue_next()
                else:
                    pl.when(g < HC - 1)(_issue_next)
            return carry

        lax.fori_loop(0, HC, step, 0)
        for k in range(4):
            pltpu.make_async_copy(rows[k], accsh.at[dstv.at[CHUNKS - 4 + k]],
                                  ssem[k]).wait()
        plsc.subcore_barrier()
        pltpu.sync_copy(accsh.at[pl.ds(s * rpt, rpt)],
                        out_hbm.at[c, pl.ds(s * rpt, rpt)])

    return pl.kernel(
        body,
        out_type=jax.ShapeDtypeStruct((NC, NP, F), jnp.float32),
        mesh=plsc.VectorSubcoreMesh(core_axis_name="c", subcore_axis_name="s"),
        compiler_params=pltpu.CompilerParams(use_tc_tiling_on_sc=False),
        scratch_types=[
            pltpu.VMEM((4, KC), jnp.int32),
            pltpu.VMEM((CHUNKS, KC), jnp.int32),
            pltpu.VMEM((KC, F), jnp.float32),
            pltpu.VMEM((KC, F), jnp.float32),
            pltpu.VMEM((KC, F), jnp.float32),
            pltpu.VMEM((KC, F), jnp.float32),
            pltpu.VMEM_SHARED((NP, F), jnp.float32),
        ] + [pltpu.SemaphoreType.DMA] * 12,
    )


def _scale_body(d0, d1, x, y, dinv):
    deg = d0[...] + d1[...] + 1.0
    r = lax.rsqrt(deg)
    y[...] = x[...] * r
    dinv[...] = r


def _mm_body(p0, p1, y1, dinv, w1, b1, w2, z):
    agg = (p0[...] + p1[...] + y1[...]) * dinv[...]
    h = jnp.dot(agg, w1[...], preferred_element_type=jnp.float32) + b1[...]
    h = jnp.maximum(h, 0.0)
    z[...] = jnp.dot(h, w2[...], preferred_element_type=jnp.float32) * dinv[...]


def _out_body(q0, q1, z, dinv, b2, o):
    o[...] = (q0[...] + q1[...] + z[...]) * dinv[...] + b2[...]


def kernel(x, edge_index, W1, b1, W2, b2):
    N, Fin = x.shape
    H = W1.shape[1]
    C = W2.shape[1]
    E = edge_index.shape[1]

    NP = ((N + NS * 128 - 1) // (NS * 128)) * (NS * 128)
    K1 = 64
    EW = -(-E // (NW * 4 * K)) * 4 * K
    EP = NW * EW
    C1 = EW // K1
    CHUNKS = EW // K
    BN = 1024

    pad = EP - E
    padidx = N + (jnp.arange(pad, dtype=edge_index.dtype) % (NP - N))
    srcp = jnp.concatenate([edge_index[0], padidx]).reshape(NW, EW)
    dstp = jnp.concatenate([edge_index[1], padidx]).reshape(NW, EW)
    src3 = srcp.reshape(NW, CHUNKS, K)
    dst3 = dstp.reshape(NW, CHUNKS, K)
    src3a = srcp.reshape(NW, C1, K1)
    dst3a = dstp.reshape(NW, C1, K1)
    xp = jnp.pad(x, ((0, NP - N), (0, 0)))
    rpt = NP // NS
    zrow = jnp.zeros((rpt,), jnp.float32)
    zrows1 = jnp.zeros((rpt, Fin), jnp.float32)
    zrows2 = jnp.zeros((rpt, C), jnp.float32)

    dp = _make_deg_kernel(NP, CHUNKS)(dst3, zrow).reshape(NC, NP)
    grid = (NP // BN,)
    row = lambda i: (i, 0)
    y1, dinv = pl.pallas_call(
        _scale_body,
        grid=grid,
        in_specs=[pl.BlockSpec((BN, 1), row), pl.BlockSpec((BN, 1), row),
                  pl.BlockSpec((BN, Fin), row)],
        out_specs=[pl.BlockSpec((BN, Fin), row), pl.BlockSpec((BN, 1), row)],
        out_shape=[jax.ShapeDtypeStruct((NP, Fin), jnp.float32),
                   jax.ShapeDtypeStruct((NP, 1), jnp.float32)],
    )(dp[0, :, None], dp[1, :, None], xp)

    p = _make_agg_kernel(NP, Fin, C1, K1)(y1, src3a, dst3a, zrows1)

    full = lambda i: (0, 0)
    z = pl.pallas_call(
        _mm_body,
        grid=grid,
        in_specs=[pl.BlockSpec((BN, Fin), row), pl.BlockSpec((BN, Fin), row),
                  pl.BlockSpec((BN, Fin), row), pl.BlockSpec((BN, 1), row),
                  pl.BlockSpec((Fin, H), full), pl.BlockSpec((1, H), full),
                  pl.BlockSpec((H, C), full)],
        out_specs=pl.BlockSpec((BN, C), row),
        out_shape=jax.ShapeDtypeStruct((NP, C), jnp.float32),
    )(p[0], p[1], y1, dinv, W1, b1.reshape(1, H), W2)

    q = _make_agg_kernel(NP, C, CHUNKS, K)(z, src3, dst3, zrows2)

    BO = 1000
    rowo = lambda i: (i, 0)
    out = pl.pallas_call(
        _out_body,
        grid=(N // BO,),
        in_specs=[pl.BlockSpec((BO, C), rowo), pl.BlockSpec((BO, C), rowo),
                  pl.BlockSpec((BO, C), rowo), pl.BlockSpec((BO, 1), rowo),
                  pl.BlockSpec((1, C), full)],
        out_specs=pl.BlockSpec((BO, C), rowo),
        out_shape=jax.ShapeDtypeStruct((N, C), jnp.float32),
    )(q[0], q[1], z, dinv, b2.reshape(1, C))
    return out


# speedup vs baseline: 42.2675x; 1.0038x over previous
import jax
import jax.numpy as jnp
from jax import lax
from jax.experimental import pallas as pl
from jax.experimental.pallas import tpu as pltpu
from jax.experimental.pallas import tpu_sc as plsc

NC = 2
NS = 16
NW = NC * NS
K = 128


def _make_deg_kernel(NP, CHUNKS):
    rpt = NP // NS

    def body(dst_hbm, zrow_hbm, out_hbm, dstv, onesv, accsh, ssem):
        c = lax.axis_index("c")
        s = lax.axis_index("s")
        wid = s * NC + c
        pltpu.sync_copy(dst_hbm.at[wid], dstv)
        for j in range(K // 16):
            onesv[pl.ds(j * 16, 16)] = jnp.full((16,), 1.0, jnp.float32)
        pltpu.sync_copy(zrow_hbm, accsh.at[pl.ds(s * rpt, rpt)])
        plsc.subcore_barrier()

        def fire(i, carry):
            pltpu.async_copy(onesv, accsh.at[dstv.at[i]], ssem, add=True)
            return carry

        lax.fori_loop(0, CHUNKS, fire, 0)

        def drain(i, carry):
            pltpu.make_async_copy(onesv, accsh.at[dstv.at[i]], ssem).wait()
            return carry

        lax.fori_loop(0, CHUNKS, drain, 0)
        plsc.subcore_barrier()
        pltpu.sync_copy(accsh.at[pl.ds(s * rpt, rpt)],
                        out_hbm.at[pl.ds(c * NP + s * rpt, rpt)])

    return pl.kernel(
        body,
        out_type=jax.ShapeDtypeStruct((NC * NP,), jnp.float32),
        mesh=plsc.VectorSubcoreMesh(core_axis_name="c", subcore_axis_name="s"),
        compiler_params=pltpu.CompilerParams(use_tc_tiling_on_sc=True),
        scratch_types=[
            pltpu.VMEM((CHUNKS, K), jnp.int32),
            pltpu.VMEM((K,), jnp.float32),
            pltpu.VMEM_SHARED((NP,), jnp.float32),
            pltpu.SemaphoreType.DMA,
        ],
    )


def _make_agg_kernel(NP, F, CHUNKS, KC, tc_tiling):
    rpt = NP // NS
    HC = CHUNKS // 4

    def body(y_hbm, src_hbm, dst_hbm, zrows_hbm, out_hbm,
             srcv, dstv, rows0, rows1, rows2, rows3, accsh, *sems):
        c = lax.axis_index("c")
        s = lax.axis_index("s")
        wid = s * NC + c
        rows = (rows0, rows1, rows2, rows3)
        gsem = sems[0:4]
        ssem = sems[4:8]
        isem = sems[8:12]
        jsem = sems[12:16]
        pltpu.sync_copy(zrows_hbm, accsh.at[pl.ds(s * rpt, rpt)])
        plsc.subcore_barrier()

        for k in range(4):
            pltpu.async_copy(src_hbm.at[wid, k], srcv.at[k], isem[k])
        for k in range(3):
            pltpu.async_copy(dst_hbm.at[wid, k], dstv.at[k], jsem[k])
        for k in range(3):
            pltpu.make_async_copy(src_hbm.at[wid, k], srcv.at[k],
                                  isem[k]).wait()
            pltpu.async_copy(y_hbm.at[srcv.at[k]], rows[k], gsem[k])

        def step(g, carry):
            for jj in range(4):
                j = 4 * g + jj
                q = jj
                qn = (jj + 3) % 4
                pltpu.make_async_copy(y_hbm.at[srcv.at[q]], rows[q],
                                      gsem[q]).wait()
                pltpu.make_async_copy(dst_hbm.at[wid, j], dstv.at[q],
                                      jsem[q]).wait()
                pltpu.async_copy(rows[q], accsh.at[dstv.at[q]], ssem[q],
                                 add=True)

                @pl.when(g < HC - 1)
                def _():
                    pltpu.async_copy(src_hbm.at[wid, j + 4], srcv.at[q],
                                     isem[q])

                def _issue_next(j=j, q=q, qn=qn):
                    def _ss():
                        pltpu.make_async_copy(
                            rows[qn], accsh.at[dstv.at[qn]],
                            ssem[qn]).wait()
                    if jj == 0:
                        pl.when(g > 0)(_ss)
                    else:
                        _ss()
                    pltpu.async_copy(dst_hbm.at[wid, j + 3], dstv.at[qn],
                                     jsem[qn])
                    pltpu.make_async_copy(src_hbm.at[wid, j + 3],
                                          srcv.at[qn], isem[qn]).wait()
                    pltpu.async_copy(y_hbm.at[srcv.at[qn]], rows[qn],
                                     gsem[qn])

                if jj == 0:
                    _issue_next()
                else:
                    pl.when(g < HC - 1)(_issue_next)
            return carry

        lax.fori_loop(0, HC, step, 0)
        for k in range(4):
            pltpu.make_async_copy(rows[k], accsh.at[dstv.at[k]],
                                  ssem[k]).wait()
        plsc.subcore_barrier()
        pltpu.sync_copy(accsh.at[pl.ds(s * rpt, rpt)],
                        out_hbm.at[c, pl.ds(s * rpt, rpt)])

    return pl.kernel(
        body,
        out_type=jax.ShapeDtypeStruct((NC, NP, F), jnp.float32),
        mesh=plsc.VectorSubcoreMesh(core_axis_name="c", subcore_axis_name="s"),
        compiler_params=pltpu.CompilerParams(use_tc_tiling_on_sc=tc_tiling),
        scratch_types=[
            pltpu.VMEM((4, KC), jnp.int32),
            pltpu.VMEM((4, KC), jnp.int32),
            pltpu.VMEM((KC, F), jnp.float32),
            pltpu.VMEM((KC, F), jnp.float32),
            pltpu.VMEM((KC, F), jnp.float32),
            pltpu.VMEM((KC, F), jnp.float32),
            pltpu.VMEM_SHARED((NP, F), jnp.float32),
        ] + [pltpu.SemaphoreType.DMA] * 16,
    )


def _scale_body(d0, d1, x, y, dinv):
    deg = d0[...] + d1[...] + 1.0
    r = lax.rsqrt(deg)
    y[...] = x[...] * r
    dinv[...] = r


def _mm_body(p0, p1, y1, dinv, w1, b1, w2, z):
    agg = (p0[...] + p1[...] + y1[...]) * dinv[...]
    h = jnp.dot(agg, w1[...], preferred_element_type=jnp.float32) + b1[...]
    h = jnp.maximum(h, 0.0)
    z[...] = jnp.dot(h, w2[...], preferred_element_type=jnp.float32) * dinv[...]


def _out_body(q0, q1, z, dinv, b2, o):
    o[...] = (q0[...] + q1[...] + z[...]) * dinv[...] + b2[...]


def kernel(x, edge_index, W1, b1, W2, b2):
    N, Fin = x.shape
    H = W1.shape[1]
    C = W2.shape[1]
    E = edge_index.shape[1]

    NP = ((N + NS * 128 - 1) // (NS * 128)) * (NS * 128)
    K1 = 64
    EW = -(-E // (NW * 4 * K)) * 4 * K
    EP = NW * EW
    C1 = EW // K1
    CHUNKS = EW // K
    BN = 1024

    pad = EP - E
    padidx = N + (jnp.arange(pad, dtype=edge_index.dtype) & 127)
    srcp = jnp.concatenate([edge_index[0], padidx]).reshape(NW, EW)
    dstp = jnp.concatenate([edge_index[1], padidx]).reshape(NW, EW)
    src3 = srcp.reshape(NW, CHUNKS, K)
    dst3 = dstp.reshape(NW, CHUNKS, K)
    src3a = srcp.reshape(NW, C1, K1)
    dst3a = dstp.reshape(NW, C1, K1)
    xp = jnp.pad(x, ((0, NP - N), (0, 0)))
    rpt = NP // NS
    zrow = jnp.zeros((rpt,), jnp.float32)
    zrows1 = jnp.zeros((rpt, Fin), jnp.float32)
    zrows2 = jnp.zeros((rpt, C), jnp.float32)

    dp = _make_deg_kernel(NP, CHUNKS)(dst3, zrow).reshape(NC, NP)
    grid = (NP // BN,)
    row = lambda i: (i, 0)
    y1, dinv = pl.pallas_call(
        _scale_body,
        grid=grid,
        in_specs=[pl.BlockSpec((BN, 1), row), pl.BlockSpec((BN, 1), row),
                  pl.BlockSpec((BN, Fin), row)],
        out_specs=[pl.BlockSpec((BN, Fin), row), pl.BlockSpec((BN, 1), row)],
        out_shape=[jax.ShapeDtypeStruct((NP, Fin), jnp.float32),
                   jax.ShapeDtypeStruct((NP, 1), jnp.float32)],
    )(dp[0, :, None], dp[1, :, None], xp)

    p = _make_agg_kernel(NP, Fin, C1, K1, True)(y1, src3a, dst3a, zrows1)

    full = lambda i: (0, 0)
    z = pl.pallas_call(
        _mm_body,
        grid=grid,
        in_specs=[pl.BlockSpec((BN, Fin), row), pl.BlockSpec((BN, Fin), row),
                  pl.BlockSpec((BN, Fin), row), pl.BlockSpec((BN, 1), row),
                  pl.BlockSpec((Fin, H), full), pl.BlockSpec((1, H), full),
                  pl.BlockSpec((H, C), full)],
        out_specs=pl.BlockSpec((BN, C), row),
        out_shape=jax.ShapeDtypeStruct((NP, C), jnp.float32),
    )(p[0], p[1], y1, dinv, W1, b1.reshape(1, H), W2)

    q = _make_agg_kernel(NP, C, CHUNKS, K, False)(z, src3, dst3, zrows2)

    BO = 1000
    rowo = lambda i: (i, 0)
    out = pl.pallas_call(
        _out_body,
        grid=(N // BO,),
        in_specs=[pl.BlockSpec((BO, C), rowo), pl.BlockSpec((BO, C), rowo),
                  pl.BlockSpec((BO, C), rowo), pl.BlockSpec((BO, 1), rowo),
                  pl.BlockSpec((1, C), full)],
        out_specs=pl.BlockSpec((BO, C), rowo),
        out_shape=jax.ShapeDtypeStruct((N, C), jnp.float32),
    )(q[0], q[1], z, dinv, b2.reshape(1, C))
    return out
